# baseline (speedup 1.0000x reference)
import functools

import jax
import jax.numpy as jnp
from jax import lax
from jax.experimental import pallas as pl
from jax.experimental.pallas import tpu as pltpu
from jax.experimental.pallas import tpu_sc as plsc

D = 64
SEQ = 50
LANES = 16
NVEC = D // LANES

NC = 2
NS = 16
NW = NC * NS

CB = 8
CHUNK = CB * SEQ

VB = 256


def _rsqrt16(x):
    xi = plsc.bitcast(x, jnp.int32)
    yi = jnp.int32(0x5F3759DF) - (xi >> 1)
    y = plsc.bitcast(yi, jnp.float32)
    half_x = x * 0.5
    for _ in range(3):
        y = y * (1.5 - half_x * y * y)
    return y


def _make_transpose(vocab):
    full_blocks = vocab // VB
    per_w = full_blocks // NW
    n_extra = full_blocks - per_w * NW
    tail = vocab - full_blocks * VB
    mesh = plsc.VectorSubcoreMesh(core_axis_name="c", subcore_axis_name="s")

    @functools.partial(
        pl.kernel,
        mesh=mesh,
        compiler_params=pltpu.CompilerParams(
            needs_layout_passes=False, use_tc_tiling_on_sc=True
        ),
        out_type=jax.ShapeDtypeStruct((vocab * D,), jnp.float32),
        scratch_types=[
            pltpu.VMEM((D, VB), jnp.float32),
            pltpu.VMEM((D, VB), jnp.float32),
            pltpu.VMEM((VB * D,), jnp.float32),
            pltpu.VMEM((VB * D,), jnp.float32),
            pltpu.SemaphoreType.DMA,
            pltpu.SemaphoreType.DMA,
            pltpu.SemaphoreType.DMA,
            pltpu.SemaphoreType.DMA,
        ],
    )
    def kern(tabt_hbm, tail_hbm, out_hbm,
             blk0, blk1, w0, w1, g0, g1, s0, s1):
        wid = lax.axis_index("s") * NC + lax.axis_index("c")
        blk = (blk0, blk1)
        wbuf = (w0, w1)
        gsem = (g0, g1)
        wsem = (s0, s1)
        base = wid * per_w

        d_idx = [lax.iota(jnp.int32, LANES) + k * LANES for k in range(NVEC)]

        def fire(local, buf):
            v0 = pl.multiple_of((base + local) * VB, VB)
            return pltpu.async_copy(
                tabt_hbm.at[:, pl.ds(v0, VB)], blk[buf], gsem[buf]
            )

        def transpose_block(buf, n_valid=VB):
            src = blk[buf]
            dst = wbuf[buf]

            @plsc.parallel_loop(0, n_valid, 1, unroll=4)
            def _(v):
                v_s = jnp.full((LANES,), v, jnp.int32)
                obase = v * D
                for k in range(NVEC):
                    col = plsc.load_gather(src, [d_idx[k], v_s])
                    dst[pl.ds(obase + k * LANES, LANES)] = col

        fire(0, 0)
        fire(1, 1)

        def pair(half, _):
            for sub in range(2):
                local = 2 * half + sub
                pltpu.make_async_copy(
                    tabt_hbm.at[:, pl.ds(0, VB)], blk[sub], gsem[sub]
                ).wait()

                @pl.when(half > 0)
                def _():
                    pltpu.make_async_copy(
                        wbuf[sub], out_hbm.at[pl.ds(0, VB * D)], wsem[sub]
                    ).wait()

                transpose_block(sub)
                v0 = pl.multiple_of((base + local) * VB, VB)
                pltpu.async_copy(
                    wbuf[sub], out_hbm.at[pl.ds(v0 * D, VB * D)], wsem[sub]
                )
                fire(local + 2, sub)
            return ()

        lax.fori_loop(0, per_w // 2, pair, ())
        for sub in range(2):
            pltpu.make_async_copy(
                tabt_hbm.at[:, pl.ds(0, VB)], blk[sub], gsem[sub]
            ).wait()
            pltpu.make_async_copy(
                wbuf[sub], out_hbm.at[pl.ds(0, VB * D)], wsem[sub]
            ).wait()

        @pl.when(wid < n_extra)
        def _():
            v0 = pl.multiple_of((per_w * NW + wid) * VB, VB)
            pltpu.sync_copy(tabt_hbm.at[:, pl.ds(v0, VB)], blk0)
            transpose_block(0)
            pltpu.sync_copy(wbuf[0], out_hbm.at[pl.ds(v0 * D, VB * D)])

        @pl.when(wid == NW - 1)
        def _():
            pltpu.sync_copy(tail_hbm, blk0)

            @plsc.parallel_loop(0, tail, 1, unroll=4)
            def _(v):
                v_s = jnp.full((LANES,), v, jnp.int32)
                obase = v * D
                for k in range(NVEC):
                    col = plsc.load_gather(blk0, [d_idx[k], v_s])
                    w0[pl.ds(obase + k * LANES, LANES)] = col

            pltpu.sync_copy(
                w0.at[pl.ds(0, tail * D)],
                out_hbm.at[pl.ds(full_blocks * VB * D, tail * D)],
            )

    return kern


def _make_kernel(n_batch, vocab):
    b_per_w = n_batch // NW
    n_chunks = b_per_w // CB
    mesh = plsc.VectorSubcoreMesh(core_axis_name="c", subcore_axis_name="s")

    @functools.partial(
        pl.kernel,
        mesh=mesh,
        compiler_params=pltpu.CompilerParams(
            needs_layout_passes=False, use_tc_tiling_on_sc=False
        ),
        out_type=jax.ShapeDtypeStruct((SEQ * D, n_batch), jnp.float32),
        scratch_types=[
            pltpu.VMEM((n_batch // NW * 56,), jnp.int32),
            pltpu.VMEM((CB, SEQ, D), jnp.float32),
            pltpu.VMEM((CB, SEQ, D), jnp.float32),
            pltpu.VMEM((SEQ * D, CB), jnp.float32),
            pltpu.VMEM((SEQ * D, CB), jnp.float32),
            pltpu.VMEM((SEQ * D,), jnp.float32),
            pltpu.VMEM((D,), jnp.float32),
            pltpu.VMEM((D,), jnp.float32),
            pltpu.SemaphoreType.DMA,
            pltpu.SemaphoreType.DMA,
            pltpu.SemaphoreType.DMA,
            pltpu.SemaphoreType.DMA,
        ],
    )
    def kern(idx_hbm, table_hbm, pe_hbm, scale_hbm, bias_hbm, out_hbm,
             idx_v, rows0, rows1, t0, t1, pe_v, scale_v, bias_v,
             gsem0, gsem1, wsem0, wsem1):
        wid = lax.axis_index("s") * NC + lax.axis_index("c")
        rows = (rows0, rows1)
        tbuf = (t0, t1)
        gsem = (gsem0, gsem1)
        wsem = (wsem0, wsem1)

        pltpu.sync_copy(pe_hbm, pe_v)
        pltpu.sync_copy(scale_hbm, scale_v)
        pltpu.sync_copy(bias_hbm, bias_v)
        b0 = pl.multiple_of(wid * b_per_w, 8)
        pltpu.sync_copy(idx_hbm.at[pl.ds(b0 * 56, b_per_w * 56)], idx_v)

        scale = [scale_v[pl.ds(k * LANES, LANES)] for k in range(NVEC)]
        bias = [bias_v[pl.ds(k * LANES, LANES)] for k in range(NVEC)]
        d_idx = [lax.iota(jnp.int32, LANES) + k * LANES for k in range(NVEC)]

        def stage(ci, buf):
            bb = pl.multiple_of(wid * b_per_w + ci * CB, 8)
            cps = [
                pltpu.async_copy(
                    table_hbm.at[idx_v.at[pl.ds((ci * CB + b) * 56, SEQ)]],
                    rows[buf].at[b],
                    gsem[buf],
                )
                for b in range(CB)
            ]
            return cps, bb

        def compute(buf):
            rbuf = rows[buf]
            obuf = tbuf[buf]

            @plsc.parallel_loop(0, CHUNK, 1, unroll=4)
            def _(j):
                jb = lax.div(j, SEQ)
                js = lax.rem(j, SEQ)
                row = rbuf.at[jb, js]
                pebase = js * D
                e = [
                    row[pl.ds(k * LANES, LANES)]
                    + pe_v[pl.ds(pebase + k * LANES, LANES)]
                    for k in range(NVEC)
                ]
                s = e[0] + e[1] + e[2] + e[3]
                q = e[0] * e[0] + e[1] * e[1] + e[2] * e[2] + e[3] * e[3]
                tot = jnp.sum(s)
                qtot = jnp.sum(q)
                mean = tot * (1.0 / D)
                var = qtot * (1.0 / D) - mean * mean
                inv = _rsqrt16(jnp.full((LANES,), var + 1e-5, jnp.float32))
                mean_v = jnp.full((LANES,), mean, jnp.float32)
                jb_v = jnp.full((LANES,), jb, jnp.int32)
                rowbase = js * D
                for k in range(NVEC):
                    val = (e[k] - mean_v) * inv * scale[k] + bias[k]
                    plsc.store_scatter(
                        obuf, [d_idx[k] + rowbase, jb_v], val
                    )

        pend = {0: stage(0, 0)}
        wcp = [None, None]
        for ci in range(n_chunks):
            cur = ci & 1
            nxt = 1 - cur
            if ci + 1 < n_chunks:
                if wcp[nxt] is not None:
                    wcp[nxt].wait()
                    wcp[nxt] = None
                pend[nxt] = stage(ci + 1, nxt)
            cps, bb = pend[cur]
            for cp in cps:
                cp.wait()
            compute(cur)
            wcp[cur] = pltpu.async_copy(
                tbuf[cur], out_hbm.at[:, pl.ds(bb, CB)], wsem[cur]
            )
        for w in wcp:
            if w is not None:
                w.wait()

    return kern


@jax.jit
def kernel(x, tok_embed, pe, norm_scale, norm_bias):
    b, s = x.shape
    vocab = tok_embed.shape[0]
    idx = jnp.pad(x.astype(jnp.int32), ((0, 0), (0, 56 - s))).reshape(-1)
    pe_flat = pe.reshape(-1)[: SEQ * D].astype(jnp.float32)
    tail = vocab - (vocab // VB) * VB
    tail_t = jnp.pad(
        tok_embed[vocab - tail:].astype(jnp.float32).T, ((0, 0), (0, VB - tail))
    )
    flat = _make_transpose(vocab)(tok_embed.T, tail_t)
    table = flat.reshape(vocab, D)
    out2 = _make_kernel(b, vocab)(
        idx, table, pe_flat,
        norm_scale.astype(jnp.float32), norm_bias.astype(jnp.float32),
    )
    return out2.reshape(SEQ, D, b).transpose(2, 0, 1)


# speedup vs baseline: 1.0000x; 1.0000x over previous
import functools

import jax
import jax.numpy as jnp
from jax import lax
from jax.experimental import pallas as pl
from jax.experimental.pallas import tpu as pltpu
from jax.experimental.pallas import tpu_sc as plsc

D = 64
SEQ = 50
LANES = 16
NVEC = D // LANES

NC = 2
NS = 16
NW = NC * NS

CB = 8
CHUNK = CB * SEQ

VB = 256


def _rsqrt16(x):
    xi = plsc.bitcast(x, jnp.int32)
    yi = jnp.int32(0x5F3759DF) - (xi >> 1)
    y = plsc.bitcast(yi, jnp.float32)
    half_x = x * 0.5
    for _ in range(3):
        y = y * (1.5 - half_x * y * y)
    return y


def _make_transpose(vocab):
    full_blocks = vocab // VB
    per_w = full_blocks // NW
    n_extra = full_blocks - per_w * NW
    tail = vocab - full_blocks * VB
    mesh = plsc.VectorSubcoreMesh(core_axis_name="c", subcore_axis_name="s")

    @functools.partial(
        pl.kernel,
        mesh=mesh,
        compiler_params=pltpu.CompilerParams(
            needs_layout_passes=False, use_tc_tiling_on_sc=True
        ),
        out_type=jax.ShapeDtypeStruct((vocab * D,), jnp.float32),
        scratch_types=[
            pltpu.VMEM((D, VB), jnp.float32),
            pltpu.VMEM((D, VB), jnp.float32),
            pltpu.VMEM((VB * D,), jnp.float32),
            pltpu.VMEM((VB * D,), jnp.float32),
            pltpu.SemaphoreType.DMA,
            pltpu.SemaphoreType.DMA,
            pltpu.SemaphoreType.DMA,
            pltpu.SemaphoreType.DMA,
        ],
    )
    def kern(tabt_hbm, tail_hbm, out_hbm,
             blk0, blk1, w0, w1, g0, g1, s0, s1):
        wid = lax.axis_index("s") * NC + lax.axis_index("c")
        blk = (blk0, blk1)
        wbuf = (w0, w1)
        gsem = (g0, g1)
        wsem = (s0, s1)
        base = wid * per_w

        d_idx = [lax.iota(jnp.int32, LANES) + k * LANES for k in range(NVEC)]

        def fire(local, buf):
            v0 = pl.multiple_of((base + local) * VB, VB)
            for td in range(D // 8):
                pltpu.async_copy(
                    tabt_hbm.at[pl.ds(8 * td, 8), pl.ds(v0, VB)],
                    blk[buf].at[pl.ds(8 * td, 8)],
                    gsem[buf],
                )

        def transpose_block(buf, n_valid=VB):
            src = blk[buf]
            dst = wbuf[buf]

            @plsc.parallel_loop(0, n_valid, 1, unroll=4)
            def _(v):
                v_s = jnp.full((LANES,), v, jnp.int32)
                obase = v * D
                for k in range(NVEC):
                    col = plsc.load_gather(src, [d_idx[k], v_s])
                    dst[pl.ds(obase + k * LANES, LANES)] = col

        fire(0, 0)
        fire(1, 1)

        def pair(half, _):
            for sub in range(2):
                local = 2 * half + sub
                pltpu.make_async_copy(
                    tabt_hbm.at[:, pl.ds(0, VB)], blk[sub], gsem[sub]
                ).wait()

                @pl.when(half > 0)
                def _():
                    pltpu.make_async_copy(
                        wbuf[sub], out_hbm.at[pl.ds(0, VB * D)], wsem[sub]
                    ).wait()

                transpose_block(sub)
                v0 = pl.multiple_of((base + local) * VB, VB)
                pltpu.async_copy(
                    wbuf[sub], out_hbm.at[pl.ds(v0 * D, VB * D)], wsem[sub]
                )
                fire(local + 2, sub)
            return ()

        lax.fori_loop(0, per_w // 2, pair, ())
        for sub in range(2):
            pltpu.make_async_copy(
                tabt_hbm.at[:, pl.ds(0, VB)], blk[sub], gsem[sub]
            ).wait()
            pltpu.make_async_copy(
                wbuf[sub], out_hbm.at[pl.ds(0, VB * D)], wsem[sub]
            ).wait()

        @pl.when(wid < n_extra)
        def _():
            v0 = pl.multiple_of((per_w * NW + wid) * VB, VB)
            pltpu.sync_copy(tabt_hbm.at[:, pl.ds(v0, VB)], blk0)
            transpose_block(0)
            pltpu.sync_copy(wbuf[0], out_hbm.at[pl.ds(v0 * D, VB * D)])

        @pl.when(wid == NW - 1)
        def _():
            pltpu.sync_copy(tail_hbm, blk0)

            @plsc.parallel_loop(0, tail, 1, unroll=4)
            def _(v):
                v_s = jnp.full((LANES,), v, jnp.int32)
                obase = v * D
                for k in range(NVEC):
                    col = plsc.load_gather(blk0, [d_idx[k], v_s])
                    w0[pl.ds(obase + k * LANES, LANES)] = col

            pltpu.sync_copy(
                w0.at[pl.ds(0, tail * D)],
                out_hbm.at[pl.ds(full_blocks * VB * D, tail * D)],
            )

    return kern


def _make_kernel(n_batch, vocab):
    b_per_w = n_batch // NW
    n_chunks = b_per_w // CB
    mesh = plsc.VectorSubcoreMesh(core_axis_name="c", subcore_axis_name="s")

    @functools.partial(
        pl.kernel,
        mesh=mesh,
        compiler_params=pltpu.CompilerParams(
            needs_layout_passes=False, use_tc_tiling_on_sc=False
        ),
        out_type=jax.ShapeDtypeStruct((SEQ * D, n_batch), jnp.float32),
        scratch_types=[
            pltpu.VMEM((n_batch // NW * 56,), jnp.int32),
            pltpu.VMEM((CB, SEQ, D), jnp.float32),
            pltpu.VMEM((CB, SEQ, D), jnp.float32),
            pltpu.VMEM((SEQ * D, CB), jnp.float32),
            pltpu.VMEM((SEQ * D, CB), jnp.float32),
            pltpu.VMEM((SEQ * D,), jnp.float32),
            pltpu.VMEM((D,), jnp.float32),
            pltpu.VMEM((D,), jnp.float32),
            pltpu.SemaphoreType.DMA,
            pltpu.SemaphoreType.DMA,
            pltpu.SemaphoreType.DMA,
            pltpu.SemaphoreType.DMA,
        ],
    )
    def kern(idx_hbm, table_hbm, pe_hbm, scale_hbm, bias_hbm, out_hbm,
             idx_v, rows0, rows1, t0, t1, pe_v, scale_v, bias_v,
             gsem0, gsem1, wsem0, wsem1):
        wid = lax.axis_index("s") * NC + lax.axis_index("c")
        rows = (rows0, rows1)
        tbuf = (t0, t1)
        gsem = (gsem0, gsem1)
        wsem = (wsem0, wsem1)

        pltpu.sync_copy(pe_hbm, pe_v)
        pltpu.sync_copy(scale_hbm, scale_v)
        pltpu.sync_copy(bias_hbm, bias_v)
        b0 = pl.multiple_of(wid * b_per_w, 8)
        pltpu.sync_copy(idx_hbm.at[pl.ds(b0 * 56, b_per_w * 56)], idx_v)

        scale = [scale_v[pl.ds(k * LANES, LANES)] for k in range(NVEC)]
        bias = [bias_v[pl.ds(k * LANES, LANES)] for k in range(NVEC)]
        d_idx = [lax.iota(jnp.int32, LANES) + k * LANES for k in range(NVEC)]

        def stage(ci, buf):
            bb = pl.multiple_of(wid * b_per_w + ci * CB, 8)
            cps = [
                pltpu.async_copy(
                    table_hbm.at[idx_v.at[pl.ds((ci * CB + b) * 56, SEQ)]],
                    rows[buf].at[b],
                    gsem[buf],
                )
                for b in range(CB)
            ]
            return cps, bb

        def compute(buf):
            rbuf = rows[buf]
            obuf = tbuf[buf]

            @plsc.parallel_loop(0, CHUNK, 1, unroll=4)
            def _(j):
                jb = lax.div(j, SEQ)
                js = lax.rem(j, SEQ)
                row = rbuf.at[jb, js]
                pebase = js * D
                e = [
                    row[pl.ds(k * LANES, LANES)]
                    + pe_v[pl.ds(pebase + k * LANES, LANES)]
                    for k in range(NVEC)
                ]
                s = e[0] + e[1] + e[2] + e[3]
                q = e[0] * e[0] + e[1] * e[1] + e[2] * e[2] + e[3] * e[3]
                tot = jnp.sum(s)
                qtot = jnp.sum(q)
                mean = tot * (1.0 / D)
                var = qtot * (1.0 / D) - mean * mean
                inv = _rsqrt16(jnp.full((LANES,), var + 1e-5, jnp.float32))
                mean_v = jnp.full((LANES,), mean, jnp.float32)
                jb_v = jnp.full((LANES,), jb, jnp.int32)
                rowbase = js * D
                for k in range(NVEC):
                    val = (e[k] - mean_v) * inv * scale[k] + bias[k]
                    plsc.store_scatter(
                        obuf, [d_idx[k] + rowbase, jb_v], val
                    )

        pend = {0: stage(0, 0)}
        wcp = [None, None]
        for ci in range(n_chunks):
            cur = ci & 1
            nxt = 1 - cur
            if ci + 1 < n_chunks:
                if wcp[nxt] is not None:
                    wcp[nxt].wait()
                    wcp[nxt] = None
                pend[nxt] = stage(ci + 1, nxt)
            cps, bb = pend[cur]
            for cp in cps:
                cp.wait()
            compute(cur)
            wcp[cur] = pltpu.async_copy(
                tbuf[cur], out_hbm.at[:, pl.ds(bb, CB)], wsem[cur]
            )
        for w in wcp:
            if w is not None:
                w.wait()

    return kern


@jax.jit
def kernel(x, tok_embed, pe, norm_scale, norm_bias):
    b, s = x.shape
    vocab = tok_embed.shape[0]
    idx = jnp.pad(x.astype(jnp.int32), ((0, 0), (0, 56 - s))).reshape(-1)
    pe_flat = pe.reshape(-1)[: SEQ * D].astype(jnp.float32)
    tail = vocab - (vocab // VB) * VB
    tail_t = jnp.pad(
        tok_embed[vocab - tail:].astype(jnp.float32).T, ((0, 0), (0, VB - tail))
    )
    flat = _make_transpose(vocab)(tok_embed.T, tail_t)
    table = flat.reshape(vocab, D)
    out2 = _make_kernel(b, vocab)(
        idx, table, pe_flat,
        norm_scale.astype(jnp.float32), norm_bias.astype(jnp.float32),
    )
    return out2.reshape(SEQ, D, b).transpose(2, 0, 1)


# speedup vs baseline: 2.3124x; 2.3124x over previous
import functools

import jax
import jax.numpy as jnp
from jax import lax
from jax.experimental import pallas as pl
from jax.experimental.pallas import tpu as pltpu
from jax.experimental.pallas import tpu_sc as plsc

D = 64
SEQ = 50
LANES = 16
NVEC = D // LANES

NC = 2
NS = 16
NW = NC * NS

CB = 8
CHUNK = CB * SEQ

VB = 256


def _rsqrt16(x):
    xi = plsc.bitcast(x, jnp.int32)
    yi = jnp.int32(0x5F3759DF) - (xi >> 1)
    y = plsc.bitcast(yi, jnp.float32)
    half_x = x * 0.5
    for _ in range(3):
        y = y * (1.5 - half_x * y * y)
    return y


def _make_transpose(vocab):
    full_blocks = vocab // VB
    per_w = full_blocks // NW
    n_extra = full_blocks - per_w * NW
    tail = vocab - full_blocks * VB
    mesh = plsc.VectorSubcoreMesh(core_axis_name="c", subcore_axis_name="s")

    @functools.partial(
        pl.kernel,
        mesh=mesh,
        compiler_params=pltpu.CompilerParams(
            needs_layout_passes=False, use_tc_tiling_on_sc=True
        ),
        out_type=jax.ShapeDtypeStruct((vocab * D,), jnp.float32),
        scratch_types=[
            pltpu.VMEM((D, VB), jnp.float32),
            pltpu.VMEM((D, VB), jnp.float32),
            pltpu.VMEM((VB * D,), jnp.float32),
            pltpu.VMEM((VB * D,), jnp.float32),
            pltpu.SemaphoreType.DMA,
            pltpu.SemaphoreType.DMA,
            pltpu.SemaphoreType.DMA,
            pltpu.SemaphoreType.DMA,
        ],
    )
    def kern(tabt_hbm, tail_hbm, out_hbm,
             blk0, blk1, w0, w1, g0, g1, s0, s1):
        wid = lax.axis_index("s") * NC + lax.axis_index("c")
        blk = (blk0, blk1)
        wbuf = (w0, w1)
        gsem = (g0, g1)
        wsem = (s0, s1)
        base = wid * per_w

        d_idx = [lax.iota(jnp.int32, LANES) + k * LANES for k in range(NVEC)]

        def fire(local, buf):
            v0 = pl.multiple_of((base + local) * VB, VB)
            for td in range(D // 8):
                pltpu.async_copy(
                    tabt_hbm.at[pl.ds(8 * td, 8), pl.ds(v0, VB)],
                    blk[buf].at[pl.ds(8 * td, 8)],
                    gsem[buf],
                )

        iota = lax.iota(jnp.int32, LANES)
        perm = [(iota + j) & (LANES - 1) for j in range(LANES)]

        def transpose_block(buf, n_valid=VB):
            src = blk[buf]
            dst = wbuf[buf]
            n_sub = (D // LANES) * (n_valid // LANES)

            @plsc.parallel_loop(0, n_sub, 1, unroll=2)
            def _(st):
                dg = lax.rem(st, D // LANES)
                ug = lax.div(st, D // LANES)
                d0 = dg * LANES
                u0 = ug * LANES
                drow = iota + jnp.full((LANES,), d0, jnp.int32)
                for j in range(LANES):
                    vcol = perm[j] + jnp.full((LANES,), u0, jnp.int32)
                    g = plsc.load_gather(src, [drow, vcol])
                    plsc.store_scatter(dst, [(vcol << 6) + drow], g)

        fire(0, 0)
        fire(1, 1)

        def pair(half, _):
            for sub in range(2):
                local = 2 * half + sub
                pltpu.make_async_copy(
                    tabt_hbm.at[:, pl.ds(0, VB)], blk[sub], gsem[sub]
                ).wait()

                @pl.when(half > 0)
                def _():
                    pltpu.make_async_copy(
                        wbuf[sub], out_hbm.at[pl.ds(0, VB * D)], wsem[sub]
                    ).wait()

                transpose_block(sub)
                v0 = pl.multiple_of((base + local) * VB, VB)
                pltpu.async_copy(
                    wbuf[sub], out_hbm.at[pl.ds(v0 * D, VB * D)], wsem[sub]
                )
                fire(local + 2, sub)
            return ()

        lax.fori_loop(0, per_w // 2, pair, ())
        for sub in range(2):
            pltpu.make_async_copy(
                tabt_hbm.at[:, pl.ds(0, VB)], blk[sub], gsem[sub]
            ).wait()
            pltpu.make_async_copy(
                wbuf[sub], out_hbm.at[pl.ds(0, VB * D)], wsem[sub]
            ).wait()

        @pl.when(wid < n_extra)
        def _():
            v0 = pl.multiple_of((per_w * NW + wid) * VB, VB)
            pltpu.sync_copy(tabt_hbm.at[:, pl.ds(v0, VB)], blk0)
            transpose_block(0)
            pltpu.sync_copy(wbuf[0], out_hbm.at[pl.ds(v0 * D, VB * D)])

        @pl.when(wid == NW - 1)
        def _():
            pltpu.sync_copy(tail_hbm, blk0)

            @plsc.parallel_loop(0, tail, 1, unroll=4)
            def _(v):
                v_s = jnp.full((LANES,), v, jnp.int32)
                obase = v * D
                for k in range(NVEC):
                    col = plsc.load_gather(blk0, [d_idx[k], v_s])
                    w0[pl.ds(obase + k * LANES, LANES)] = col

            pltpu.sync_copy(
                w0.at[pl.ds(0, tail * D)],
                out_hbm.at[pl.ds(full_blocks * VB * D, tail * D)],
            )

    return kern


def _make_kernel(n_batch, vocab):
    b_per_w = n_batch // NW
    n_chunks = b_per_w // CB
    mesh = plsc.VectorSubcoreMesh(core_axis_name="c", subcore_axis_name="s")

    @functools.partial(
        pl.kernel,
        mesh=mesh,
        compiler_params=pltpu.CompilerParams(
            needs_layout_passes=False, use_tc_tiling_on_sc=False
        ),
        out_type=jax.ShapeDtypeStruct((SEQ * D, n_batch), jnp.float32),
        scratch_types=[
            pltpu.VMEM((n_batch // NW * 56,), jnp.int32),
            pltpu.VMEM((CB, SEQ, D), jnp.float32),
            pltpu.VMEM((CB, SEQ, D), jnp.float32),
            pltpu.VMEM((SEQ * D, CB), jnp.float32),
            pltpu.VMEM((SEQ * D, CB), jnp.float32),
            pltpu.VMEM((SEQ * D,), jnp.float32),
            pltpu.VMEM((D,), jnp.float32),
            pltpu.VMEM((D,), jnp.float32),
            pltpu.SemaphoreType.DMA,
            pltpu.SemaphoreType.DMA,
            pltpu.SemaphoreType.DMA,
            pltpu.SemaphoreType.DMA,
        ],
    )
    def kern(idx_hbm, table_hbm, pe_hbm, scale_hbm, bias_hbm, out_hbm,
             idx_v, rows0, rows1, t0, t1, pe_v, scale_v, bias_v,
             gsem0, gsem1, wsem0, wsem1):
        wid = lax.axis_index("s") * NC + lax.axis_index("c")
        rows = (rows0, rows1)
        tbuf = (t0, t1)
        gsem = (gsem0, gsem1)
        wsem = (wsem0, wsem1)

        pltpu.sync_copy(pe_hbm, pe_v)
        pltpu.sync_copy(scale_hbm, scale_v)
        pltpu.sync_copy(bias_hbm, bias_v)
        b0 = pl.multiple_of(wid * b_per_w, 8)
        pltpu.sync_copy(idx_hbm.at[pl.ds(b0 * 56, b_per_w * 56)], idx_v)

        scale = [scale_v[pl.ds(k * LANES, LANES)] for k in range(NVEC)]
        bias = [bias_v[pl.ds(k * LANES, LANES)] for k in range(NVEC)]
        d_idx = [lax.iota(jnp.int32, LANES) + k * LANES for k in range(NVEC)]

        def stage(ci, buf):
            bb = pl.multiple_of(wid * b_per_w + ci * CB, 8)
            cps = [
                pltpu.async_copy(
                    table_hbm.at[idx_v.at[pl.ds((ci * CB + b) * 56, SEQ)]],
                    rows[buf].at[b],
                    gsem[buf],
                )
                for b in range(CB)
            ]
            return cps, bb

        def compute(buf):
            rbuf = rows[buf]
            obuf = tbuf[buf]

            @plsc.parallel_loop(0, CHUNK, 1, unroll=4)
            def _(j):
                jb = lax.div(j, SEQ)
                js = lax.rem(j, SEQ)
                row = rbuf.at[jb, js]
                pebase = js * D
                e = [
                    row[pl.ds(k * LANES, LANES)]
                    + pe_v[pl.ds(pebase + k * LANES, LANES)]
                    for k in range(NVEC)
                ]
                s = e[0] + e[1] + e[2] + e[3]
                q = e[0] * e[0] + e[1] * e[1] + e[2] * e[2] + e[3] * e[3]
                tot = jnp.sum(s)
                qtot = jnp.sum(q)
                mean = tot * (1.0 / D)
                var = qtot * (1.0 / D) - mean * mean
                inv = _rsqrt16(jnp.full((LANES,), var + 1e-5, jnp.float32))
                mean_v = jnp.full((LANES,), mean, jnp.float32)
                jb_v = jnp.full((LANES,), jb, jnp.int32)
                rowbase = js * D
                for k in range(NVEC):
                    val = (e[k] - mean_v) * inv * scale[k] + bias[k]
                    plsc.store_scatter(
                        obuf, [d_idx[k] + rowbase, jb_v], val
                    )

        pend = {0: stage(0, 0)}
        wcp = [None, None]
        for ci in range(n_chunks):
            cur = ci & 1
            nxt = 1 - cur
            if ci + 1 < n_chunks:
                if wcp[nxt] is not None:
                    wcp[nxt].wait()
                    wcp[nxt] = None
                pend[nxt] = stage(ci + 1, nxt)
            cps, bb = pend[cur]
            for cp in cps:
                cp.wait()
            compute(cur)
            wcp[cur] = pltpu.async_copy(
                tbuf[cur], out_hbm.at[:, pl.ds(bb, CB)], wsem[cur]
            )
        for w in wcp:
            if w is not None:
                w.wait()

    return kern


@jax.jit
def kernel(x, tok_embed, pe, norm_scale, norm_bias):
    b, s = x.shape
    vocab = tok_embed.shape[0]
    idx = jnp.pad(x.astype(jnp.int32), ((0, 0), (0, 56 - s))).reshape(-1)
    pe_flat = pe.reshape(-1)[: SEQ * D].astype(jnp.float32)
    tail = vocab - (vocab // VB) * VB
    tail_t = jnp.pad(
        tok_embed[vocab - tail:].astype(jnp.float32).T, ((0, 0), (0, VB - tail))
    )
    flat = _make_transpose(vocab)(tok_embed.T, tail_t)
    table = flat.reshape(vocab, D)
    out2 = _make_kernel(b, vocab)(
        idx, table, pe_flat,
        norm_scale.astype(jnp.float32), norm_bias.astype(jnp.float32),
    )
    return out2.reshape(SEQ, D, b).transpose(2, 0, 1)
